# baseline (speedup 1.0000x reference)
import dataclasses

import jax
import jax.numpy as jnp
from jax import lax
from jax.experimental import pallas as pl
from jax.experimental.pallas import tpu as pltpu
from jax.experimental.pallas import tpu_sc as plsc

BATCH = 16384
N_FIELDS = 26
NC = 2
NS = 16
NW = NC * NS
B_PER_W = BATCH // NW
IDX_PER_W = B_PER_W * N_FIELDS
IDX_MINOR = 128
ROWS_PER_F = B_PER_W // IDX_MINOR
LANES = 16
TABLE_LEN = 1000000
TABLE_PAD = 1000448
SPLIT = (TABLE_LEN // 1024) * 1024


def _sc_body(w_hbm, idx_hbm, out_hbm, idx_v, vals_v, out_v, sem):
    wid = lax.axis_index("s") * NC + lax.axis_index("c")
    base = wid * B_PER_W

    pltpu.sync_copy(idx_hbm.at[:, wid], idx_v)

    @pl.loop(0, N_FIELDS)
    def _(f):
        for q in range(ROWS_PER_F):
            pltpu.async_copy(
                w_hbm.at[idx_v.at[f, q]],
                vals_v.at[pl.ds(f * B_PER_W + q * IDX_MINOR, IDX_MINOR)],
                sem,
            )
    pltpu.make_async_copy(w_hbm.at[pl.ds(0, IDX_PER_W)], vals_v, sem).wait()

    for j0 in range(0, B_PER_W, LANES):
        acc = vals_v[pl.ds(j0, LANES)]
        for f in range(1, N_FIELDS):
            acc = acc + vals_v[pl.ds(f * B_PER_W + j0, LANES)]
        out_v[pl.ds(j0, LANES)] = acc

    pltpu.sync_copy(out_v, out_hbm.at[pl.ds(base, B_PER_W)])


@jax.jit
def _sc_call(w_flat, idx_t):
    mesh = plsc.VectorSubcoreMesh(core_axis_name="c", subcore_axis_name="s")
    cp = pltpu.CompilerParams()
    fields = pltpu.CompilerParams.__dataclass_fields__
    if "needs_layout_passes" in fields:
        cp = dataclasses.replace(cp, needs_layout_passes=False)
    if "use_tc_tiling_on_sc" in fields:
        cp = dataclasses.replace(cp, use_tc_tiling_on_sc=False)
    run = pl.kernel(
        _sc_body,
        compiler_params=cp,
        out_type=jax.ShapeDtypeStruct((BATCH,), jnp.float32),
        mesh=mesh,
        scratch_types=[
            pltpu.VMEM((N_FIELDS, ROWS_PER_F, IDX_MINOR), jnp.int32),
            pltpu.VMEM((IDX_PER_W,), jnp.float32),
            pltpu.VMEM((B_PER_W,), jnp.float32),
            pltpu.SemaphoreType.DMA,
        ],
    )
    return run(w_flat, idx_t)


def kernel(inputs, w):
    idx = inputs.astype(jnp.int32).T.reshape(N_FIELDS, NW, ROWS_PER_F, IDX_MINOR)
    p1 = w[:SPLIT, :].reshape(-1)
    p2 = w[SPLIT:, :].reshape(-1)
    w_flat = lax.dynamic_update_slice(
        jnp.pad(p1, (0, TABLE_PAD - SPLIT)), p2, (SPLIT,)
    )
    out = _sc_call(w_flat, idx)
    return out.reshape(BATCH, 1)


# speedup vs baseline: 1.0332x; 1.0332x over previous
import dataclasses

import jax
import jax.numpy as jnp
from jax import lax
from jax.experimental import pallas as pl
from jax.experimental.pallas import tpu as pltpu
from jax.experimental.pallas import tpu_sc as plsc

BATCH = 16384
N_FIELDS = 26
NC = 2
NS = 16
NW = NC * NS
B_PER_W = BATCH // NW
IDX_PER_W = B_PER_W * N_FIELDS
IDX_MINOR = 128
ROWS_PER_F = B_PER_W // IDX_MINOR
LANES = 16
TABLE_LEN = 1000000
TABLE_PAD = 1000448
SPLIT = (TABLE_LEN // 1024) * 1024


def _sc_body(w_hbm, idx_hbm, out_hbm, idx_v, vals_v, out_v, sem):
    wid = lax.axis_index("s") * NC + lax.axis_index("c")
    base = wid * B_PER_W

    pltpu.sync_copy(idx_hbm.at[:, wid], idx_v)

    @pl.loop(0, N_FIELDS)
    def _(f):
        for q in range(ROWS_PER_F):
            pltpu.async_copy(
                w_hbm.at[idx_v.at[f, q]],
                vals_v.at[pl.ds(f * B_PER_W + q * IDX_MINOR, IDX_MINOR)],
                sem,
            )
    pltpu.make_async_copy(w_hbm.at[pl.ds(0, IDX_PER_W)], vals_v, sem).wait()

    @pl.loop(0, B_PER_W, step=2 * LANES)
    def _(j0):
        for s in range(2):
            acc = vals_v[pl.ds(j0 + s * LANES, LANES)]
            for f in range(1, N_FIELDS):
                acc = acc + vals_v[pl.ds(f * B_PER_W + j0 + s * LANES, LANES)]
            out_v[pl.ds(j0 + s * LANES, LANES)] = acc

    pltpu.sync_copy(out_v, out_hbm.at[pl.ds(base, B_PER_W)])


@jax.jit
def _sc_call(w_flat, idx_t):
    mesh = plsc.VectorSubcoreMesh(core_axis_name="c", subcore_axis_name="s")
    cp = pltpu.CompilerParams()
    fields = pltpu.CompilerParams.__dataclass_fields__
    if "needs_layout_passes" in fields:
        cp = dataclasses.replace(cp, needs_layout_passes=False)
    if "use_tc_tiling_on_sc" in fields:
        cp = dataclasses.replace(cp, use_tc_tiling_on_sc=False)
    run = pl.kernel(
        _sc_body,
        compiler_params=cp,
        out_type=jax.ShapeDtypeStruct((BATCH,), jnp.float32),
        mesh=mesh,
        scratch_types=[
            pltpu.VMEM((N_FIELDS, ROWS_PER_F, IDX_MINOR), jnp.int32),
            pltpu.VMEM((IDX_PER_W,), jnp.float32),
            pltpu.VMEM((B_PER_W,), jnp.float32),
            pltpu.SemaphoreType.DMA,
        ],
    )
    return run(w_flat, idx_t)


def kernel(inputs, w):
    idx = inputs.astype(jnp.int32).T.reshape(N_FIELDS, NW, ROWS_PER_F, IDX_MINOR)
    p1 = w[:SPLIT, :].reshape(-1)
    p2 = w[SPLIT:, :].reshape(-1)
    w_flat = lax.dynamic_update_slice(
        jnp.pad(p1, (0, TABLE_PAD - SPLIT)), p2, (SPLIT,)
    )
    out = _sc_call(w_flat, idx)
    return out.reshape(BATCH, 1)


# speedup vs baseline: 1.0509x; 1.0171x over previous
import dataclasses

import jax
import jax.numpy as jnp
from jax import lax
from jax.experimental import pallas as pl
from jax.experimental.pallas import tpu as pltpu
from jax.experimental.pallas import tpu_sc as plsc

BATCH = 16384
N_FIELDS = 26
NC = 2
NS = 16
NW = NC * NS
B_PER_W = BATCH // NW
IDX_PER_W = B_PER_W * N_FIELDS
IDX_MINOR = 128
ROWS_PER_F = B_PER_W // IDX_MINOR
LANES = 16
TABLE_LEN = 1000000
TABLE_PAD = 1000448
SPLIT = (TABLE_LEN // 1024) * 1024


def _sc_body(w_hbm, idx_hbm, out_hbm, idx_v, vals_v, out_v, sem):
    wid = lax.axis_index("s") * NC + lax.axis_index("c")
    base = wid * B_PER_W

    pltpu.sync_copy(idx_hbm.at[:, wid], idx_v)

    @pl.loop(0, N_FIELDS)
    def _(f):
        for q in range(ROWS_PER_F):
            pltpu.async_copy(
                w_hbm.at[idx_v.at[f, q]],
                vals_v.at[pl.ds(f * B_PER_W + q * IDX_MINOR, IDX_MINOR)],
                sem,
            )
    pltpu.make_async_copy(w_hbm.at[pl.ds(0, IDX_PER_W)], vals_v, sem).wait()

    @pl.loop(0, B_PER_W, step=2 * LANES)
    def _(j0):
        for s in range(2):
            acc = vals_v[pl.ds(j0 + s * LANES, LANES)]
            for f in range(1, N_FIELDS):
                acc = acc + vals_v[pl.ds(f * B_PER_W + j0 + s * LANES, LANES)]
            out_v[pl.ds(j0 + s * LANES, LANES)] = acc

    pltpu.sync_copy(out_v, out_hbm.at[pl.ds(base, B_PER_W)])


@jax.jit
def _sc_call(w_flat, idx_t):
    mesh = plsc.VectorSubcoreMesh(core_axis_name="c", subcore_axis_name="s")
    cp = pltpu.CompilerParams()
    fields = pltpu.CompilerParams.__dataclass_fields__
    if "needs_layout_passes" in fields:
        cp = dataclasses.replace(cp, needs_layout_passes=False)
    if "use_tc_tiling_on_sc" in fields:
        cp = dataclasses.replace(cp, use_tc_tiling_on_sc=False)
    run = pl.kernel(
        _sc_body,
        compiler_params=cp,
        out_type=jax.ShapeDtypeStruct((BATCH,), jnp.float32),
        mesh=mesh,
        scratch_types=[
            pltpu.VMEM((N_FIELDS, ROWS_PER_F, IDX_MINOR), jnp.int32),
            pltpu.VMEM((IDX_PER_W,), jnp.float32),
            pltpu.VMEM((B_PER_W,), jnp.float32),
            pltpu.SemaphoreType.DMA,
        ],
    )
    return run(w_flat, idx_t)


def kernel(inputs, w):
    idx = inputs.astype(jnp.int32).T.reshape(N_FIELDS, NW, ROWS_PER_F, IDX_MINOR)
    p1 = w[:SPLIT, :].reshape(-1)
    p2 = lax.optimization_barrier(w[SPLIT:, :]).reshape(-1)
    w_flat = lax.dynamic_update_slice(
        jnp.pad(p1, (0, TABLE_PAD - SPLIT)), p2, (SPLIT,)
    )
    out = _sc_call(w_flat, idx)
    return out.reshape(BATCH, 1)


# speedup vs baseline: 1.0959x; 1.0428x over previous
import dataclasses

import jax
import jax.numpy as jnp
from jax import lax
from jax.experimental import pallas as pl
from jax.experimental.pallas import tpu as pltpu
from jax.experimental.pallas import tpu_sc as plsc

BATCH = 16384
N_FIELDS = 26
NC = 2
NS = 16
NW = NC * NS
B_PER_W = BATCH // NW
IDX_PER_W = B_PER_W * N_FIELDS
IDX_MINOR = 128
ROWS_PER_F = B_PER_W // IDX_MINOR
LANES = 16
TABLE_LEN = 1000000
TABLE_PAD = 1000448
SPLIT = (TABLE_LEN // 1024) * 1024


def _sc_body(w_hbm, idx_hbm, out_hbm, idx_v, vals_v, out_v, sem):
    wid = lax.axis_index("s") * NC + lax.axis_index("c")
    base = wid * B_PER_W

    pltpu.sync_copy(idx_hbm.at[:, wid], idx_v)

    @pl.loop(0, N_FIELDS)
    def _(f):
        for q in range(ROWS_PER_F):
            pltpu.async_copy(
                w_hbm.at[idx_v.at[f, q]],
                vals_v.at[pl.ds(f * B_PER_W + q * IDX_MINOR, IDX_MINOR)],
                sem,
            )
    pltpu.make_async_copy(w_hbm.at[pl.ds(0, IDX_PER_W)], vals_v, sem).wait()

    @pl.loop(0, B_PER_W, step=2 * LANES)
    def _(j0):
        for s in range(2):
            acc = vals_v[pl.ds(j0 + s * LANES, LANES)]
            for f in range(1, N_FIELDS):
                acc = acc + vals_v[pl.ds(f * B_PER_W + j0 + s * LANES, LANES)]
            out_v[pl.ds(j0 + s * LANES, LANES)] = acc

    pltpu.sync_copy(out_v, out_hbm.at[pl.ds(base, B_PER_W)])


@jax.jit
def _sc_call(w_flat, idx_t):
    mesh = plsc.VectorSubcoreMesh(core_axis_name="c", subcore_axis_name="s")
    cp = pltpu.CompilerParams()
    fields = pltpu.CompilerParams.__dataclass_fields__
    if "needs_layout_passes" in fields:
        cp = dataclasses.replace(cp, needs_layout_passes=False)
    if "use_tc_tiling_on_sc" in fields:
        cp = dataclasses.replace(cp, use_tc_tiling_on_sc=False)
    run = pl.kernel(
        _sc_body,
        compiler_params=cp,
        out_type=jax.ShapeDtypeStruct((BATCH,), jnp.float32),
        mesh=mesh,
        scratch_types=[
            pltpu.VMEM((N_FIELDS, ROWS_PER_F, IDX_MINOR), jnp.int32),
            pltpu.VMEM((IDX_PER_W,), jnp.float32),
            pltpu.VMEM((B_PER_W,), jnp.float32),
            pltpu.SemaphoreType.DMA,
        ],
    )
    return run(w_flat, idx_t)


def kernel(inputs, w):
    idx = inputs.astype(jnp.int32).T.reshape(N_FIELDS, NW, ROWS_PER_F, IDX_MINOR)
    half = SPLIT // 2
    p1a = w[:half, :].reshape(-1)
    p1b = lax.optimization_barrier(w[half:SPLIT, :]).reshape(-1)
    p2 = lax.optimization_barrier(w[SPLIT:, :]).reshape(-1)
    w_flat = lax.dynamic_update_slice(
        lax.dynamic_update_slice(
            jnp.pad(p1a, (0, TABLE_PAD - half)), p1b, (half,)
        ),
        p2, (SPLIT,),
    )
    out = _sc_call(w_flat, idx)
    return out.reshape(BATCH, 1)
